# baseline (speedup 1.0000x reference)
import functools

import jax
import jax.numpy as jnp
import numpy as np
from jax import lax
from jax.experimental import pallas as pl
from jax.experimental.pallas import tpu as pltpu
from jax.experimental.pallas import tpu_sc as plsc

_TIMESTEPS = 1000
_BATCH = 16384
_TABLE = _TIMESTEPS + 1
_LANES = 16
_NUM_CORES = 2
_NUM_SUBCORES = 16
_NUM_WORKERS = _NUM_CORES * _NUM_SUBCORES
_CHUNK = _BATCH // _NUM_WORKERS
_MAGIC = np.float32(1.5 * 2.0**23)


def _body(t_hbm, gamma_hbm, out_hbm, gamma_v, t_v, o_v, sem_g, sem_t):
    wid = lax.axis_index("s") * _NUM_CORES + lax.axis_index("c")
    base = wid * _CHUNK
    cp_g = pltpu.async_copy(gamma_hbm, gamma_v, sem_g)
    cp_t = pltpu.async_copy(t_hbm.at[pl.ds(base, _CHUNK)], t_v, sem_t)
    cp_g.wait()
    cp_t.wait()
    for i in range(_CHUNK // _LANES):
        tv = t_v[pl.ds(i * _LANES, _LANES)]
        x = tv * np.float32(_TIMESTEPS)
        s = x + _MAGIC
        idx = lax.bitcast_convert_type(s, jnp.int32) - np.int32(0x4B400000)
        o_v[pl.ds(i * _LANES, _LANES)] = plsc.load_gather(gamma_v, [idx])
    pltpu.sync_copy(o_v, out_hbm.at[pl.ds(base, _CHUNK)])


@jax.jit
def kernel(t, gamma):
    mesh = plsc.VectorSubcoreMesh(core_axis_name="c", subcore_axis_name="s")
    k = functools.partial(
        pl.kernel,
        mesh=mesh,
        out_type=jax.ShapeDtypeStruct((_BATCH,), jnp.float32),
        scratch_types=[
            pltpu.VMEM((_TABLE,), jnp.float32),
            pltpu.VMEM((_CHUNK,), jnp.float32),
            pltpu.VMEM((_CHUNK,), jnp.float32),
            pltpu.SemaphoreType.DMA,
            pltpu.SemaphoreType.DMA,
        ],
        compiler_params=pltpu.CompilerParams(needs_layout_passes=False),
    )(_body)
    return k(t, gamma)


# speedup vs baseline: 1.0864x; 1.0864x over previous
import functools

import jax
import jax.numpy as jnp
import numpy as np
from jax import lax
from jax.experimental import pallas as pl
from jax.experimental.pallas import tpu as pltpu
from jax.experimental.pallas import tpu_sc as plsc

_TIMESTEPS = 1000
_BATCH = 16384
_TABLE = _TIMESTEPS + 1
_LANES = 16
_NUM_CORES = 1
_NUM_SUBCORES = 16
_NUM_WORKERS = _NUM_CORES * _NUM_SUBCORES
_CHUNK = _BATCH // _NUM_WORKERS
_MAGIC = np.float32(1.5 * 2.0**23)


def _body(t_hbm, gamma_hbm, out_hbm, gamma_v, t_v, o_v, sem_g, sem_t):
    wid = lax.axis_index("s") * _NUM_CORES + lax.axis_index("c")
    base = wid * _CHUNK
    cp_g = pltpu.async_copy(gamma_hbm, gamma_v, sem_g)
    cp_t = pltpu.async_copy(t_hbm.at[pl.ds(base, _CHUNK)], t_v, sem_t)
    cp_g.wait()
    cp_t.wait()
    for i in range(_CHUNK // _LANES):
        tv = t_v[pl.ds(i * _LANES, _LANES)]
        x = tv * np.float32(_TIMESTEPS)
        s = x + _MAGIC
        idx = lax.bitcast_convert_type(s, jnp.int32) - np.int32(0x4B400000)
        o_v[pl.ds(i * _LANES, _LANES)] = plsc.load_gather(gamma_v, [idx])
    pltpu.sync_copy(o_v, out_hbm.at[pl.ds(base, _CHUNK)])


@jax.jit
def kernel(t, gamma):
    mesh = plsc.VectorSubcoreMesh(
        core_axis_name="c", subcore_axis_name="s", num_cores=_NUM_CORES
    )
    k = functools.partial(
        pl.kernel,
        mesh=mesh,
        out_type=jax.ShapeDtypeStruct((_BATCH,), jnp.float32),
        scratch_types=[
            pltpu.VMEM((_TABLE,), jnp.float32),
            pltpu.VMEM((_CHUNK,), jnp.float32),
            pltpu.VMEM((_CHUNK,), jnp.float32),
            pltpu.SemaphoreType.DMA,
            pltpu.SemaphoreType.DMA,
        ],
        compiler_params=pltpu.CompilerParams(needs_layout_passes=False),
    )(_body)
    return k(t, gamma)
